# baseline (speedup 1.0000x reference)
import functools

import jax
import jax.numpy as jnp
from jax import lax
from jax.experimental import pallas as pl
from jax.experimental.pallas import tpu as pltpu
from jax.experimental.pallas import tpu_sc as plsc


@functools.lru_cache(maxsize=None)
def _build(batch: int, hist: int, vocab: int, embed: int):
    info = plsc.get_sparse_core_info()
    nc, ns, nl = info.num_cores, info.num_subcores, info.num_lanes
    nw = nc * ns
    assert batch % (16 * nw) == 0 and hist % 8 == 0 and embed == 64
    n_hblk = hist // 8
    n_bblk = batch // 128
    assert n_bblk == nw
    eg = embed // nl

    mesh = plsc.VectorSubcoreMesh(core_axis_name="c", subcore_axis_name="s")

    @functools.partial(
        pl.kernel,
        mesh=mesh,
        out_type=jax.ShapeDtypeStruct((hist, embed // 8, n_bblk, 1024),
                                      jnp.float32),
        scratch_types=[
            pltpu.VMEM((8, 128), jnp.int32),
            [pltpu.VMEM((128, embed), jnp.float32) for _ in range(2)],
            [pltpu.VMEM((8 * 1024,), jnp.float32) for _ in range(2)],
            [pltpu.SemaphoreType.DMA for _ in range(2)],
            [pltpu.SemaphoreType.DMA for _ in range(2)],
        ],
        compiler_params=pltpu.CompilerParams(use_tc_tiling_on_sc=False,
                                             needs_layout_passes=False),
    )
    def k(idx_hbm, tbl_hbm, out_hbm, idx_v, rows_v, asm_v, sem_g, sem_o):
        w = lax.axis_index("s") * nc + lax.axis_index("c")
        base_iota = lax.iota(jnp.int32, nl) * 128

        def do_hblk(hblk, carry):
            pltpu.sync_copy(idx_hbm.at[hblk, w], idx_v)
            pltpu.async_copy(tbl_hbm.at[idx_v.at[0]], rows_v[0], sem_g[0])

            for sh in range(8):
                b = sh % 2
                if sh < 7:
                    pltpu.async_copy(tbl_hbm.at[idx_v.at[sh + 1]],
                                     rows_v[(sh + 1) % 2], sem_g[(sh + 1) % 2])
                pltpu.make_async_copy(tbl_hbm.at[idx_v.at[sh]], rows_v[b],
                                      sem_g[b]).wait()

                @pl.when(jnp.logical_or(hblk > 0, sh >= 2))
                def _():
                    for t in range(embed // 8):
                        pltpu.make_async_copy(
                            asm_v[b].at[pl.ds(t * 1024, 1024)],
                            out_hbm.at[0, t, w], sem_o[b]).wait()

                def do_row(j, sidx):
                    for g in range(eg):
                        val = rows_v[b][j, pl.ds(g * nl, nl)]
                        plsc.store_scatter(asm_v[b], [sidx[g]], val)
                    return tuple(s + 1 for s in sidx)

                lax.fori_loop(
                    0, 128, do_row,
                    tuple(base_iota + g * nl * 128 for g in range(eg)),
                    unroll=2)

                h = hblk * 8 + sh
                for t in range(embed // 8):
                    pltpu.async_copy(asm_v[b].at[pl.ds(t * 1024, 1024)],
                                     out_hbm.at[h, t, w], sem_o[b])
            return carry

        lax.fori_loop(0, n_hblk, do_hblk, 0)

        for b in range(2):
            for t in range(embed // 8):
                pltpu.make_async_copy(asm_v[b].at[pl.ds(t * 1024, 1024)],
                                      out_hbm.at[0, t, w], sem_o[b]).wait()

    return k


def kernel(input_tensor, embedding_table):
    batch, hist = input_tensor.shape
    vocab, embed = embedding_table.shape
    idx4 = (input_tensor.T.reshape(hist // 8, 8, batch // 128, 128)
            .transpose(0, 2, 1, 3))
    out5 = _build(batch, hist, vocab, embed)(idx4, embedding_table)
    out = (out5.reshape(hist, embed // 8, batch // 128, 8, 128)
           .transpose(2, 4, 0, 1, 3).reshape(batch, hist, embed))
    return out
